# Initial kernel scaffold; baseline (speedup 1.0000x reference)
import functools

import jax
import jax.numpy as jnp
from jax.experimental import pallas as pl
from jax.experimental.pallas import tpu as pltpu
from jax.experimental.pallas import tpu_sc as plsc

_WINDOW = 512


def kernel(style_idx, table):
    batch, hist = style_idx.shape
    num_rows, embed_dim = table.shape
    n = batch * hist

    idx = style_idx.reshape(1, n).astype(jnp.int32)
    mesh = plsc.VectorSubcoreMesh(core_axis_name="c", subcore_axis_name="s")

    @functools.partial(
        pl.kernel,
        out_type=jax.ShapeDtypeStruct((n, embed_dim), table.dtype),
        mesh=mesh,
    )
    def gather_kernel(table_hbm, idx_hbm, out_hbm):
        def body(i_vmem, o_vmem):
            pltpu.sync_copy(table_hbm.at[i_vmem.at[0]], o_vmem)

        pltpu.emit_pipeline(
            body,
            grid=(n // _WINDOW,),
            in_specs=[pl.BlockSpec((1, _WINDOW), index_map=lambda i: (0, i))],
            out_specs=[
                pl.BlockSpec((_WINDOW, embed_dim), index_map=lambda i: (i, 0))
            ],
            core_axis_name=("c", "s"),
            dimension_semantics=(pltpu.PARALLEL,),
        )(idx_hbm, out_hbm)

    out = gather_kernel(table, idx)
    return out.reshape(batch, hist, embed_dim)


# speedup vs baseline: 6.2088x; 6.2088x over previous
import functools

import jax
import jax.numpy as jnp
from jax.experimental import pallas as pl
from jax.experimental.pallas import tpu as pltpu
from jax.experimental.pallas import tpu_sc as plsc

_WINDOW = 512


def kernel(style_idx, table):
    batch, hist = style_idx.shape
    num_rows, embed_dim = table.shape
    n = batch * hist

    idx = style_idx.reshape(1, n).astype(jnp.int32)
    mesh = plsc.VectorSubcoreMesh(core_axis_name="c", subcore_axis_name="s")

    @functools.partial(
        pl.kernel,
        out_type=jax.ShapeDtypeStruct((n, embed_dim), table.dtype),
        mesh=mesh,
        compiler_params=pltpu.CompilerParams(use_tc_tiling_on_sc=False),
    )
    def gather_kernel(table_hbm, idx_hbm, out_hbm):
        def body(i_vmem, o_vmem):
            pltpu.sync_copy(table_hbm.at[i_vmem.at[0]], o_vmem)

        pltpu.emit_pipeline(
            body,
            grid=(n // _WINDOW,),
            in_specs=[pl.BlockSpec((1, _WINDOW), index_map=lambda i: (0, i))],
            out_specs=[
                pl.BlockSpec((_WINDOW, embed_dim), index_map=lambda i: (i, 0))
            ],
            core_axis_name=("c", "s"),
            dimension_semantics=(pltpu.PARALLEL,),
        )(idx_hbm, out_hbm)

    out = gather_kernel(table, idx)
    return out.reshape(batch, hist, embed_dim)


# speedup vs baseline: 6.2510x; 1.0068x over previous
import functools

import jax
import jax.numpy as jnp
from jax import lax
from jax.experimental import pallas as pl
from jax.experimental.pallas import tpu as pltpu
from jax.experimental.pallas import tpu_sc as plsc

_NUM_WORKERS = 32
_GROUP = 8
_NBUF = 2


def kernel(style_idx, table):
    batch, hist = style_idx.shape
    num_rows, embed_dim = table.shape
    n = batch * hist

    idx = style_idx.reshape(1, n).astype(jnp.int32)
    mesh = plsc.VectorSubcoreMesh(core_axis_name="c", subcore_axis_name="s")

    per_w = batch // _NUM_WORKERS
    ch = _GROUP * hist
    steps = per_w // _GROUP

    @functools.partial(
        pl.kernel,
        out_type=jax.ShapeDtypeStruct((batch, hist, embed_dim), table.dtype),
        mesh=mesh,
        compiler_params=pltpu.CompilerParams(use_tc_tiling_on_sc=False),
        scratch_types=[
            pltpu.VMEM((1, per_w * hist), jnp.int32),
            pltpu.VMEM((_NBUF, ch, embed_dim), jnp.float32),
            pltpu.SemaphoreType.DMA,
            pltpu.SemaphoreType.DMA,
            pltpu.SemaphoreType.DMA,
            pltpu.SemaphoreType.DMA,
        ],
    )
    def gather_kernel(table_hbm, idx_hbm, out_hbm, idx_v, rows_v, g0, g1, w0, w1):
        gsem = [g0, g1]
        wsem = [w0, w1]
        wid = lax.axis_index("s") * 2 + lax.axis_index("c")
        b0 = wid * per_w

        pltpu.sync_copy(idx_hbm.at[0, pl.ds(b0 * hist, per_w * hist)], idx_v.at[0])

        def issue_gather(k, slot):
            pltpu.async_copy(
                table_hbm.at[idx_v.at[0, pl.ds(k * ch, ch)]],
                rows_v.at[slot],
                gsem[slot],
            )

        def wait_gather(slot):
            pltpu.make_async_copy(
                table_hbm.at[pl.ds(0, ch)], rows_v.at[slot], gsem[slot]
            ).wait()

        def issue_writes(k, slot):
            for j in range(_GROUP):
                pltpu.async_copy(
                    rows_v.at[slot, pl.ds(j * hist, hist)],
                    out_hbm.at[b0 + k * _GROUP + j],
                    wsem[slot],
                )

        def drain_writes(slot):
            for _ in range(_GROUP):
                pltpu.make_async_copy(
                    rows_v.at[slot, pl.ds(0, hist)], out_hbm.at[0], wsem[slot]
                ).wait()

        issue_gather(0, 0)

        @pl.loop(0, steps, step=_NBUF)
        def _(t):
            for b in range(_NBUF):
                k = t + b
                nslot = (b + 1) % _NBUF

                @pl.when(k + 1 < steps)
                def _prefetch():
                    @pl.when(k + 1 >= _NBUF)
                    def _drain():
                        drain_writes(nslot)

                    issue_gather(k + 1, nslot)

                wait_gather(b)
                issue_writes(k, b)

        for b in range(_NBUF):
            drain_writes(b)

    return gather_kernel(table, idx)


# speedup vs baseline: 12.7029x; 2.0321x over previous
import functools

import jax
import jax.numpy as jnp
from jax import lax
from jax.experimental import pallas as pl
from jax.experimental.pallas import tpu as pltpu
from jax.experimental.pallas import tpu_sc as plsc

_NUM_WORKERS = 32
_GROUP = 8
_NBUF = 2
_TC_BATCH_BLOCK = 512


def kernel(style_idx, table):
    batch, hist = style_idx.shape
    num_rows, embed_dim = table.shape
    n = batch * hist

    idx = style_idx.reshape(1, n).astype(jnp.int32)
    mesh = plsc.VectorSubcoreMesh(core_axis_name="c", subcore_axis_name="s")

    per_w = batch // _NUM_WORKERS
    ch = _GROUP * hist
    steps = per_w // _GROUP

    @functools.partial(
        pl.kernel,
        out_type=jax.ShapeDtypeStruct((n, embed_dim), table.dtype),
        mesh=mesh,
        compiler_params=pltpu.CompilerParams(use_tc_tiling_on_sc=False),
        scratch_types=[
            pltpu.VMEM((1, per_w * hist), jnp.int32),
            pltpu.VMEM((_NBUF, ch, embed_dim), jnp.float32),
            pltpu.SemaphoreType.DMA,
            pltpu.SemaphoreType.DMA,
            pltpu.SemaphoreType.DMA,
            pltpu.SemaphoreType.DMA,
        ],
    )
    def gather_kernel(table_hbm, idx_hbm, out_hbm, idx_v, rows_v, g0, g1, w0, w1):
        gsem = [g0, g1]
        wsem = [w0, w1]
        wid = lax.axis_index("s") * 2 + lax.axis_index("c")
        b0 = wid * per_w

        pltpu.sync_copy(idx_hbm.at[0, pl.ds(b0 * hist, per_w * hist)], idx_v.at[0])

        def issue_gather(k, slot):
            pltpu.async_copy(
                table_hbm.at[idx_v.at[0, pl.ds(k * ch, ch)]],
                rows_v.at[slot],
                gsem[slot],
            )

        def wait_gather(slot):
            pltpu.make_async_copy(
                table_hbm.at[pl.ds(0, ch)], rows_v.at[slot], gsem[slot]
            ).wait()

        def issue_writes(k, slot):
            pltpu.async_copy(
                rows_v.at[slot],
                out_hbm.at[pl.ds((b0 + k * _GROUP) * hist, ch)],
                wsem[slot],
            )

        def drain_writes(slot):
            pltpu.make_async_copy(
                rows_v.at[slot], out_hbm.at[pl.ds(0, ch)], wsem[slot]
            ).wait()

        issue_gather(0, 0)

        @pl.loop(0, steps, step=_NBUF)
        def _(t):
            for b in range(_NBUF):
                k = t + b
                nslot = (b + 1) % _NBUF

                @pl.when(k + 1 < steps)
                def _prefetch():
                    @pl.when(k + 1 >= _NBUF)
                    def _drain():
                        drain_writes(nslot)

                    issue_gather(k + 1, nslot)

                wait_gather(b)
                issue_writes(k, b)

        for b in range(_NBUF):
            drain_writes(b)

    gathered = gather_kernel(table, idx)

    row = hist * embed_dim
    rpb = row // 128
    in2d = gathered.reshape(n * embed_dim // 128, 128)

    bblk = _TC_BATCH_BLOCK

    def transpose_body(in_ref, out_ref):
        for r in range(rpb):
            out_ref[pl.ds(r * 128, 128), :] = in_ref[r::rpb, :].T

    xt = pl.pallas_call(
        transpose_body,
        out_shape=jax.ShapeDtypeStruct((row, batch), table.dtype),
        grid=(batch // bblk,),
        in_specs=[pl.BlockSpec((bblk * rpb, 128), lambda j: (j, 0))],
        out_specs=pl.BlockSpec((row, bblk), lambda j: (0, j)),
        compiler_params=pltpu.CompilerParams(
            dimension_semantics=("parallel",)
        ),
    )(in2d)

    x3 = xt.reshape(hist, embed_dim, batch)
    return jnp.transpose(x3, (2, 0, 1))
